# baseline (speedup 1.0000x reference)
import math
import functools

import jax
import jax.numpy as jnp
from jax import lax
from jax.experimental import pallas as pl
from jax.experimental.pallas import tpu as pltpu
from jax.experimental.pallas import tpu_sc as plsc

_T = 512
_E = 16
_CAP = int(math.ceil(_T / _E * 1.05))
_CAPP = 40
_S = _E * _CAPP
_BUFROWS = _S + _CAPP
_GROWS = _S + _CAPP * 2
_DUMP = _S
_NW = 32
_TPT = _T // _NW


def _router_body(x_ref, wg_ref, slot_ref, slotc_ref, gs_ref):
    x = x_ref[...]
    T = x.shape[0]
    E = wg_ref.shape[1]
    logits = jnp.dot(x.astype(jnp.bfloat16), wg_ref[...].astype(jnp.bfloat16),
                     preferred_element_type=jnp.float32)
    m = jnp.max(logits, axis=-1, keepdims=True)
    ex = jnp.exp(logits - m)
    probs = ex / jnp.sum(ex, axis=-1, keepdims=True)
    g = jnp.max(probs, axis=-1)
    e_idx = jnp.argmax(probs, axis=-1).astype(jnp.int32)

    iota_e = jax.lax.broadcasted_iota(jnp.int32, (T, E), 1)
    oh = (iota_e == e_idx[:, None]).astype(jnp.float32)
    c = oh
    k = 1
    while k < T:
        c = c + jnp.concatenate(
            [jnp.zeros((k, E), jnp.float32), c[:-k]], axis=0)
        k *= 2
    pos = jnp.sum(c * oh, axis=-1).astype(jnp.int32) - 1
    keep = pos < _CAP
    slot = jnp.where(keep, e_idx * _CAPP + pos, _DUMP)
    slot_ref[...] = slot
    slotc_ref[...] = jnp.minimum(slot, _S - 1)

    row_iota = jax.lax.broadcasted_iota(jnp.int32, (_GROWS, T), 0)
    P = (row_iota == slot[None, :]).astype(jnp.bfloat16)
    g16 = jnp.broadcast_to(g[:, None], (T, 16))
    gh = g16.astype(jnp.bfloat16)
    gl = (g16 - gh.astype(jnp.float32)).astype(jnp.bfloat16)
    gs_ref[...] = (jnp.dot(P, gh, preferred_element_type=jnp.float32)
                   + jnp.dot(P, gl, preferred_element_type=jnp.float32))


def _dispatch_body(slot_hbm, x_hbm, buf_hbm, slot_v, xv_v, sem):
    w = lax.axis_index("s") * 2 + lax.axis_index("c")
    pltpu.sync_copy(x_hbm.at[pl.ds(w * _TPT, _TPT)], xv_v)
    pltpu.sync_copy(slot_hbm.at[pl.ds(w * _TPT, _TPT)], slot_v)
    pltpu.async_copy(xv_v, buf_hbm.at[slot_v], sem).wait()


def _combine_body(y_hbm, slotc_hbm, out_hbm, sv_v, rows_v, sem):
    w = lax.axis_index("s") * 2 + lax.axis_index("c")
    pltpu.sync_copy(slotc_hbm.at[pl.ds(w * _TPT, _TPT)], sv_v)
    pltpu.async_copy(y_hbm.at[sv_v], rows_v, sem).wait()
    pltpu.sync_copy(rows_v, out_hbm.at[pl.ds(w * _TPT, _TPT)])


def _ffn_body(nd, nf, DB, FB, buf_ref, w1_ref, b1_ref, w2_ref, b2_ref, g_ref,
              y_ref, h_ref):
    j = pl.program_id(1)

    @pl.when(j < nd)
    def _():
        a_sl = buf_ref[:, pl.ds(j * DB, DB)]
        part = jnp.dot(a_sl, w1_ref[0], preferred_element_type=jnp.float32)

        @pl.when(j == 0)
        def _():
            h_ref[...] = part + b1_ref[0]

        @pl.when(j > 0)
        def _():
            h_ref[...] += part

    @pl.when(j >= nd)
    def _():
        fi = j - nd
        hs = jax.nn.gelu(h_ref[:, pl.ds(fi * FB, FB)])
        contrib = jnp.dot(hs, w2_ref[0], preferred_element_type=jnp.float32)

        @pl.when(j == nd)
        def _():
            y_ref[...] = contrib + b2_ref[0]

        @pl.when(j > nd)
        def _():
            y_ref[...] += contrib

        @pl.when(j == nd + nf - 1)
        def _():
            gcol = g_ref[:, 0:1]
            y = y_ref[...]
            y_ref[...] = jnp.where(gcol > 0.0, y * gcol, 0.0)


def kernel(x, Wg, W1, b1, W2, b2):
    T, D = x.shape
    E = Wg.shape[1]
    F = W1.shape[2]
    FB = 1024
    nf = F // FB
    DB = 256
    nd = D // DB

    slot, slotc, gslot = pl.pallas_call(
        _router_body,
        out_shape=(
            jax.ShapeDtypeStruct((T,), jnp.int32),
            jax.ShapeDtypeStruct((T,), jnp.int32),
            jax.ShapeDtypeStruct((_GROWS, 16), jnp.float32),
        ),
    )(x, Wg)

    dispatch = pl.kernel(
        _dispatch_body,
        out_type=jax.ShapeDtypeStruct((_BUFROWS, D), jnp.float32),
        mesh=plsc.VectorSubcoreMesh(core_axis_name="c", subcore_axis_name="s",
                                    num_cores=2, num_subcores=16),
        scratch_types=[
            pltpu.VMEM((_TPT,), jnp.int32),
            pltpu.VMEM((_TPT, D), jnp.float32),
            pltpu.SemaphoreType.DMA,
        ],
    )
    buf = dispatch(slot, x)

    y = pl.pallas_call(
        functools.partial(_ffn_body, nd, nf, DB, FB),
        grid=(E, nd + nf),
        in_specs=[
            pl.BlockSpec((_CAPP, D), lambda e, j: (e, 0)),
            pl.BlockSpec((1, DB, F), lambda e, j: (e, jnp.minimum(j, nd - 1), 0)),
            pl.BlockSpec((1, 1, F), lambda e, j: (e, 0, 0)),
            pl.BlockSpec((1, FB, D), lambda e, j: (e, jnp.maximum(j - nd, 0), 0)),
            pl.BlockSpec((1, 1, D), lambda e, j: (e, 0, 0)),
            pl.BlockSpec((_CAPP, 16), lambda e, j: (e, 0)),
        ],
        out_specs=pl.BlockSpec((_CAPP, D), lambda e, j: (e, 0)),
        out_shape=jax.ShapeDtypeStruct((_S, D), jnp.float32),
        scratch_shapes=[pltpu.VMEM((_CAPP, F), jnp.float32)],
    )(buf, W1, b1.reshape(E, 1, F), W2, b2.reshape(E, 1, D), gslot)

    combine = pl.kernel(
        _combine_body,
        out_type=jax.ShapeDtypeStruct((T, D), jnp.float32),
        mesh=plsc.VectorSubcoreMesh(core_axis_name="c", subcore_axis_name="s",
                                    num_cores=2, num_subcores=16),
        scratch_types=[
            pltpu.VMEM((_TPT,), jnp.int32),
            pltpu.VMEM((_TPT, D), jnp.float32),
            pltpu.SemaphoreType.DMA,
        ],
    )
    return combine(y, slotc)


# speedup vs baseline: 1.3735x; 1.3735x over previous
import math
import functools

import jax
import jax.numpy as jnp
from jax import lax
from jax.experimental import pallas as pl
from jax.experimental.pallas import tpu as pltpu
from jax.experimental.pallas import tpu_sc as plsc

_T = 512
_E = 16
_CAP = int(math.ceil(_T / _E * 1.05))
_CAPP = 40
_S = _E * _CAPP
_BUFROWS = _S + _CAPP
_GROWS = _S + _CAPP * 2
_DUMP = _S
_NW = 32
_TPT = _T // _NW


def _router_body(x_ref, wg_ref, slot_ref, slotc_ref, gs_ref):
    x = x_ref[...]
    T = x.shape[0]
    E = wg_ref.shape[1]
    logits = jnp.dot(x.astype(jnp.bfloat16), wg_ref[...].astype(jnp.bfloat16),
                     preferred_element_type=jnp.float32)
    m = jnp.max(logits, axis=-1, keepdims=True)
    ex = jnp.exp(logits - m)
    probs = ex / jnp.sum(ex, axis=-1, keepdims=True)
    g = jnp.max(probs, axis=-1)
    e_idx = jnp.argmax(probs, axis=-1).astype(jnp.int32)

    iota_e = jax.lax.broadcasted_iota(jnp.int32, (T, E), 1)
    oh = (iota_e == e_idx[:, None]).astype(jnp.float32)
    c = oh
    k = 1
    while k < T:
        c = c + jnp.concatenate(
            [jnp.zeros((k, E), jnp.float32), c[:-k]], axis=0)
        k *= 2
    pos = jnp.sum(c * oh, axis=-1).astype(jnp.int32) - 1
    keep = pos < _CAP
    slot = jnp.where(keep, e_idx * _CAPP + pos, _DUMP)
    slot_ref[...] = slot
    slotc_ref[...] = jnp.minimum(slot, _S - 1)

    row_iota = jax.lax.broadcasted_iota(jnp.int32, (_GROWS, T), 0)
    P = (row_iota == slot[None, :]).astype(jnp.bfloat16)
    g16 = jnp.broadcast_to(g[:, None], (T, 16))
    gh = g16.astype(jnp.bfloat16)
    gl = (g16 - gh.astype(jnp.float32)).astype(jnp.bfloat16)
    gs_ref[...] = (jnp.dot(P, gh, preferred_element_type=jnp.float32)
                   + jnp.dot(P, gl, preferred_element_type=jnp.float32))


def _dispatch_body(slot_hbm, x_hbm, buf_hbm, slot_v, xv_v, sem):
    w = lax.axis_index("s") * 2 + lax.axis_index("c")
    pltpu.sync_copy(x_hbm.at[pl.ds(w * _TPT, _TPT)], xv_v)
    pltpu.sync_copy(slot_hbm.at[pl.ds(w * _TPT, _TPT)], slot_v)
    pltpu.async_copy(xv_v, buf_hbm.at[slot_v], sem).wait()


def _combine_body(y_hbm, slotc_hbm, out_hbm, sv_v, rows_v, sem):
    w = lax.axis_index("s") * 2 + lax.axis_index("c")
    pltpu.sync_copy(slotc_hbm.at[pl.ds(w * _TPT, _TPT)], sv_v)
    pltpu.async_copy(y_hbm.at[sv_v], rows_v, sem).wait()
    pltpu.sync_copy(rows_v, out_hbm.at[pl.ds(w * _TPT, _TPT)])


def _ffn_body(nf, buf_ref, w1_ref, b1_ref, w2_ref, b2_ref, g_ref, y_ref):
    f = pl.program_id(1)
    a = buf_ref[...]
    h = jnp.dot(a, w1_ref[0], preferred_element_type=jnp.float32) + b1_ref[0]
    h = jax.nn.gelu(h)
    contrib = jnp.dot(h, w2_ref[0], preferred_element_type=jnp.float32)

    @pl.when(f == 0)
    def _():
        y_ref[...] = contrib + b2_ref[0]

    @pl.when(f > 0)
    def _():
        y_ref[...] += contrib

    @pl.when(f == nf - 1)
    def _():
        gcol = g_ref[:, 0:1]
        y = y_ref[...]
        y_ref[...] = jnp.where(gcol > 0.0, y * gcol, 0.0)


def kernel(x, Wg, W1, b1, W2, b2):
    T, D = x.shape
    E = Wg.shape[1]
    F = W1.shape[2]
    FB = 1536
    nf = F // FB

    slot, slotc, gslot = pl.pallas_call(
        _router_body,
        out_shape=(
            jax.ShapeDtypeStruct((T,), jnp.int32),
            jax.ShapeDtypeStruct((T,), jnp.int32),
            jax.ShapeDtypeStruct((_GROWS, 16), jnp.float32),
        ),
    )(x, Wg)

    dispatch = pl.kernel(
        _dispatch_body,
        out_type=jax.ShapeDtypeStruct((_BUFROWS, D), jnp.float32),
        mesh=plsc.VectorSubcoreMesh(core_axis_name="c", subcore_axis_name="s",
                                    num_cores=2, num_subcores=16),
        scratch_types=[
            pltpu.VMEM((_TPT,), jnp.int32),
            pltpu.VMEM((_TPT, D), jnp.float32),
            pltpu.SemaphoreType.DMA,
        ],
    )
    buf = dispatch(slot, x)

    y = pl.pallas_call(
        functools.partial(_ffn_body, nf),
        grid=(E, nf),
        in_specs=[
            pl.BlockSpec((_CAPP, D), lambda e, f: (e, 0)),
            pl.BlockSpec((1, D, FB), lambda e, f: (e, 0, f)),
            pl.BlockSpec((1, 1, FB), lambda e, f: (e, 0, f)),
            pl.BlockSpec((1, FB, D), lambda e, f: (e, f, 0)),
            pl.BlockSpec((1, 1, D), lambda e, f: (e, 0, 0)),
            pl.BlockSpec((_CAPP, 16), lambda e, f: (e, 0)),
        ],
        out_specs=pl.BlockSpec((_CAPP, D), lambda e, f: (e, 0)),
        out_shape=jax.ShapeDtypeStruct((_S, D), jnp.float32),
    )(buf, W1, b1.reshape(E, 1, F), W2, b2.reshape(E, 1, D), gslot)

    combine = pl.kernel(
        _combine_body,
        out_type=jax.ShapeDtypeStruct((T, D), jnp.float32),
        mesh=plsc.VectorSubcoreMesh(core_axis_name="c", subcore_axis_name="s",
                                    num_cores=2, num_subcores=16),
        scratch_types=[
            pltpu.VMEM((_TPT,), jnp.int32),
            pltpu.VMEM((_TPT, D), jnp.float32),
            pltpu.SemaphoreType.DMA,
        ],
    )
    return combine(y, slotc)
